# baseline (speedup 1.0000x reference)
import functools

import jax
import jax.numpy as jnp
from jax import lax
from jax.experimental import pallas as pl
from jax.experimental.pallas import tpu as pltpu
from jax.experimental.pallas import tpu_sc as plsc

_L = 16


def _make_sc_kernel(B, S, C, n_workers, rows_per_worker):
    full_chunks = C // _L
    tail = C - full_chunks * _L
    C_pad = C if tail == 0 else (full_chunks + 1) * _L
    n_len_chunks = B // _L

    mesh = plsc.VectorSubcoreMesh(core_axis_name="c", subcore_axis_name="s")

    @functools.partial(
        pl.kernel,
        out_type=jax.ShapeDtypeStruct((B, C), jnp.float32),
        mesh=mesh,
        scratch_types=[
            pltpu.VMEM((B,), jnp.int32),
            pltpu.VMEM((B,), jnp.int32),
            pltpu.VMEM((rows_per_worker, C), jnp.float32),
            pltpu.VMEM((rows_per_worker, C_pad), jnp.float32),
            pltpu.VMEM((C,), jnp.float32),
            pltpu.VMEM((C,), jnp.float32),
            pltpu.SemaphoreType.DMA,
        ],
        compiler_params=pltpu.CompilerParams(use_tc_tiling_on_sc=False),
    )
    def sc_kernel(x_hbm, len_hbm, gamma_hbm, beta_hbm, out_hbm,
                  len_v, idx_v, rows_v, out_v, gam_v, bet_v, sem):
        wid = lax.axis_index("s") * 2 + lax.axis_index("c")

        @pl.when(wid < n_workers)
        def _():
            pltpu.sync_copy(len_hbm, len_v)
            pltpu.sync_copy(gamma_hbm.at[0], gam_v)
            pltpu.sync_copy(beta_hbm.at[0], bet_v)

            lane = jax.lax.iota(jnp.int32, _L)
            for k in range(n_len_chunks):
                lv = len_v[pl.ds(k * _L, _L)]
                seq_row = lax.rem(lv + (S - 1), S)
                idx_v[pl.ds(k * _L, _L)] = seq_row + (lane + k * _L) * S

            base = wid * rows_per_worker
            pltpu.async_copy(
                x_hbm.at[idx_v.at[pl.ds(base, rows_per_worker)]],
                rows_v, sem,
            ).wait()

            for r in range(rows_per_worker):
                def chunk(i, _):
                    v = rows_v[r, pl.ds(i * _L, _L)]
                    g = gam_v[pl.ds(i * _L, _L)]
                    bta = bet_v[pl.ds(i * _L, _L)]
                    out_v[r, pl.ds(i * _L, _L)] = v * g + bta
                    return 0
                lax.fori_loop(0, full_chunks, chunk, 0, unroll=False)
                if tail:
                    off = C - _L
                    v = rows_v[r, pl.ds(off, _L)]
                    g = gam_v[pl.ds(off, _L)]
                    bta = bet_v[pl.ds(off, _L)]
                    out_v[r, pl.ds(off, _L)] = v * g + bta

            for r in range(rows_per_worker):
                pltpu.sync_copy(out_v.at[r, pl.ds(0, C)],
                                out_hbm.at[base + r])

    return sc_kernel


@jax.jit
def kernel(x, length, gamma, beta):
    B, S, C = x.shape
    n_workers = 8
    rows_per_worker = B // n_workers
    sc = _make_sc_kernel(B, S, C, n_workers, rows_per_worker)
    out = sc(x.reshape(B * S, C), length.astype(jnp.int32), gamma, beta)
    return out[:, None, :]


# speedup vs baseline: 2.4251x; 2.4251x over previous
import jax
import jax.numpy as jnp
from jax import lax
from jax.experimental import pallas as pl
from jax.experimental.pallas import tpu as pltpu


def _make_body(S):
    def body(len_ref, x_ref, gamma_ref, beta_ref, out_ref):
        b = pl.program_id(0)
        row = lax.rem(len_ref[b] + (S - 1), S)
        sub = lax.rem(row, 8)
        sel = lax.broadcasted_iota(jnp.int32, (8, 1), 0) == sub
        picked = jnp.sum(jnp.where(sel, x_ref[...], 0.0), axis=0,
                         keepdims=True)
        out_ref[...] = (picked * gamma_ref[...] + beta_ref[...])[:, None, :]

    return body


@jax.jit
def kernel(x, length, gamma, beta):
    B, S, C = x.shape

    def x_index(b, len_ref):
        row = lax.rem(len_ref[b] + (S - 1), S)
        return (b * (S // 8) + lax.div(row, 8), 0)

    grid_spec = pltpu.PrefetchScalarGridSpec(
        num_scalar_prefetch=1,
        grid=(B,),
        in_specs=[
            pl.BlockSpec((8, C), x_index),
            pl.BlockSpec((1, C), lambda b, len_ref: (0, 0)),
            pl.BlockSpec((1, C), lambda b, len_ref: (0, 0)),
        ],
        out_specs=pl.BlockSpec((1, 1, C), lambda b, len_ref: (b, 0, 0)),
    )

    out = pl.pallas_call(
        _make_body(S),
        grid_spec=grid_spec,
        out_shape=jax.ShapeDtypeStruct((B, 1, C), jnp.float32),
    )(length.astype(jnp.int32), x.reshape(B * S, C), gamma, beta)
    return out
